# baseline (speedup 1.0000x reference)
import jax
import jax.numpy as jnp
from jax import lax
from jax.experimental import pallas as pl
from jax.experimental.pallas import tpu as pltpu
from jax.experimental.pallas import tpu_sc as plsc

B = 4
C = 32
D = H = W = 64
N = 100000
NTOT = B * N

NC = 2
NS = 16
NW = NC * NS
L = 16

DHW = D * H * W
VTOT = B * DHW

BLK = 1024
VOXW = VTOT // NW
NBLK = VOXW // BLK

CV = 128
NCHT = NTOT // CV
NCH_HI = -(-NCHT // NW)
NCH_LO = NCHT // NW
N_HI = NCHT - NCH_LO * NW

_CORNER_OFFS = (0, W, H * W, H * W + W, 1, W + 1, H * W + 1, H * W + W + 1)


def _axis01(v, extent):
    t = (v + 1.0) * 0.5 * (extent - 1)
    t = jnp.clip(t, 0.0, float(extent - 1))
    i0 = jnp.minimum(t.astype(jnp.int32), extent - 2)
    frac = t - i0.astype(jnp.float32)
    return i0, frac


def _relayout_body(vol, table, lines_v, tbuf_v, sem_in, sem_out):
    wid = lax.axis_index("s") * NC + lax.axis_index("c")
    b = wid // (NW // B)
    loc0 = (wid % (NW // B)) * VOXW

    def blk_body(blk, _):
        off = loc0 + blk * BLK
        cpy = [
            pltpu.async_copy(
                vol.at[pl.ds((b * C + c) * DHW + off, BLK)],
                lines_v.at[c], sem_in)
            for c in range(C)
        ]
        for cp in cpy:
            cp.wait()

        lanes = lax.iota(jnp.int32, L)

        def vox_body(v, _):
            lo = plsc.load_gather(lines_v, [lanes, jnp.broadcast_to(v, (L,))])
            hi = plsc.load_gather(
                lines_v, [lanes + L, jnp.broadcast_to(v, (L,))])
            tbuf_v[v, pl.ds(0, L)] = lo
            tbuf_v[v, pl.ds(L, L)] = hi
            return 0

        lax.fori_loop(0, BLK, vox_body, 0)
        pltpu.async_copy(
            tbuf_v, table.at[pl.ds(b * DHW + off, BLK)], sem_out).wait()
        return 0

    lax.fori_loop(0, NBLK, blk_body, 0)


def _sample_body(table, verts, out, idx_v, rows_v, wgt_v, crd_v, outb_v,
                 sem_g, sem_c):
    wid = lax.axis_index("s") * NC + lax.axis_index("c")

    def chunk_body(ch, _):
        vbase = ch * CV
        pltpu.async_copy(verts.at[pl.ds(vbase * 3, CV * 3)], crd_v,
                         sem_c).wait()

        lanes = lax.iota(jnp.int32, L)
        for g in range(CV // L):
            tri = (g * L + lanes) * 3
            xv = plsc.load_gather(crd_v, [tri])
            yv = plsc.load_gather(crd_v, [tri + 1])
            zv = plsc.load_gather(crd_v, [tri + 2])
            gid = vbase + g * L + lanes
            one = jnp.ones((L,), jnp.int32)
            zero = jnp.zeros((L,), jnp.int32)
            bid = (jnp.where(gid >= N, one, zero)
                   + jnp.where(gid >= 2 * N, one, zero)
                   + jnp.where(gid >= 3 * N, one, zero))
            x0, wx = _axis01(xv, W)
            y0, wy = _axis01(yv, H)
            z0, wz = _axis01(zv, D)
            rbase = (bid << 18) + (z0 << 12) + (y0 << 6) + x0
            for k in range(8):
                idx_v[k, pl.ds(g * L, L)] = rbase + _CORNER_OFFS[k]
            wy1 = wy
            wy0 = 1.0 - wy
            wz1 = wz
            wz0 = 1.0 - wz
            wgt_v[pl.ds(0 * CV + g * L, L)] = wx
            wgt_v[pl.ds(1 * CV + g * L, L)] = wz0 * wy0
            wgt_v[pl.ds(2 * CV + g * L, L)] = wz0 * wy1
            wgt_v[pl.ds(3 * CV + g * L, L)] = wz1 * wy0
            wgt_v[pl.ds(4 * CV + g * L, L)] = wz1 * wy1

        gcopies = [
            pltpu.async_copy(table.at[idx_v.at[k]], rows_v.at[k], sem_g)
            for k in range(8)
        ]
        for cp in gcopies:
            cp.wait()

        def lerp_group(g, _):
            wxv = wgt_v[pl.ds(0 * CV + g * L, L)]
            w00v = wgt_v[pl.ds(1 * CV + g * L, L)]
            w01v = wgt_v[pl.ds(2 * CV + g * L, L)]
            w10v = wgt_v[pl.ds(3 * CV + g * L, L)]
            w11v = wgt_v[pl.ds(4 * CV + g * L, L)]
            for lane in range(L):
                i = g * L + lane
                wx = wxv[lane]
                w00 = w00v[lane]
                w01 = w01v[lane]
                w10 = w10v[lane]
                w11 = w11v[lane]
                for h in range(C // L):
                    hs = pl.ds(h * L, L)
                    a = (w00 * rows_v[0, i, hs] + w01 * rows_v[1, i, hs]
                         + w10 * rows_v[2, i, hs] + w11 * rows_v[3, i, hs])
                    bb = (w00 * rows_v[4, i, hs] + w01 * rows_v[5, i, hs]
                          + w10 * rows_v[6, i, hs] + w11 * rows_v[7, i, hs])
                    outb_v[pl.ds(i * C + h * L, L)] = a + wx * (bb - a)
            return 0

        lax.fori_loop(0, CV // L, lerp_group, 0)
        pltpu.sync_copy(outb_v, out.at[pl.ds(vbase * C, CV * C)])
        return 0

    def loop_lo(j, _):
        return chunk_body(j * NW + wid, 0)

    lax.fori_loop(0, NCH_LO, loop_lo, 0)

    @pl.when(wid < N_HI)
    def _():
        chunk_body(NCH_LO * NW + wid, 0)


_MESH = plsc.VectorSubcoreMesh(core_axis_name="c", subcore_axis_name="s")
_PARAMS = pltpu.CompilerParams(
    use_tc_tiling_on_sc=False, needs_layout_passes=False)


@jax.jit
def _run(vol_flat, verts_flat):
    table = pl.kernel(
        _relayout_body,
        out_type=jax.ShapeDtypeStruct((VTOT, C), jnp.float32),
        mesh=_MESH,
        compiler_params=_PARAMS,
        scratch_types=[
            pltpu.VMEM((C, BLK), jnp.float32),
            pltpu.VMEM((BLK, C), jnp.float32),
            pltpu.SemaphoreType.DMA,
            pltpu.SemaphoreType.DMA,
        ],
    )(vol_flat)
    out = pl.kernel(
        _sample_body,
        out_type=jax.ShapeDtypeStruct((NTOT * C,), jnp.float32),
        mesh=_MESH,
        compiler_params=_PARAMS,
        scratch_types=[
            pltpu.VMEM((8, CV), jnp.int32),
            pltpu.VMEM((8, CV, C), jnp.float32),
            pltpu.VMEM((5 * CV,), jnp.float32),
            pltpu.VMEM((3 * CV,), jnp.float32),
            pltpu.VMEM((CV * C,), jnp.float32),
            pltpu.SemaphoreType.DMA,
            pltpu.SemaphoreType.DMA,
        ],
    )(table, verts_flat)
    return out.reshape(B, N, C)


def kernel(voxel_features, vertices, pad_img_shape):
    del pad_img_shape
    return _run(voxel_features.reshape(-1), vertices.reshape(-1))


# speedup vs baseline: 1.1969x; 1.1969x over previous
import jax
import jax.numpy as jnp
from jax import lax
from jax.experimental import pallas as pl
from jax.experimental.pallas import tpu as pltpu
from jax.experimental.pallas import tpu_sc as plsc

B = 4
C = 32
D = H = W = 64
N = 100000
NTOT = B * N

NC = 2
NS = 16
NW = NC * NS
L = 16

DHW = D * H * W
VTOT = B * DHW

CV = 128
NCHT = NTOT // CV
NCH_LO = NCHT // NW
N_HI = NCHT - NCH_LO * NW

_CORNER_OFFS = (0, W, H * W, H * W + W, 1, W + 1, H * W + 1, H * W + W + 1)


def _axis01(v, extent):
    t = (v + 1.0) * 0.5 * (extent - 1)
    t = jnp.clip(t, 0.0, float(extent - 1))
    i0 = jnp.minimum(t.astype(jnp.int32), extent - 2)
    frac = t - i0.astype(jnp.float32)
    return i0, frac


def _sample_body(table, verts, out, idx_v, rows_v, wgt_v, crd_v, outb_v,
                 sem_g, sem_c):
    wid = lax.axis_index("s") * NC + lax.axis_index("c")
    nch = NCH_LO + jnp.where(wid < N_HI, 1, 0)

    def fire_coords(j):
        ch = j * NW + wid
        pltpu.async_copy(verts.at[pl.ds(ch * CV, CV)], crd_v.at[j % 2],
                         sem_c)

    def phase_a(j):
        s = j % 2
        ch = j * NW + wid
        vbase = ch * CV
        pltpu.make_async_copy(verts.at[pl.ds(vbase, CV)], crd_v.at[s],
                              sem_c).wait()
        lanes = lax.iota(jnp.int32, L)
        for g in range(CV // L):
            rows16 = g * L + lanes
            col0 = jnp.zeros((L,), jnp.int32)
            xv = plsc.load_gather(crd_v, [jnp.full((L,), s), rows16, col0])
            yv = plsc.load_gather(crd_v, [jnp.full((L,), s), rows16,
                                          col0 + 1])
            zv = plsc.load_gather(crd_v, [jnp.full((L,), s), rows16,
                                          col0 + 2])
            gid = vbase + g * L + lanes
            one = jnp.ones((L,), jnp.int32)
            zero = jnp.zeros((L,), jnp.int32)
            bid = (jnp.where(gid >= N, one, zero)
                   + jnp.where(gid >= 2 * N, one, zero)
                   + jnp.where(gid >= 3 * N, one, zero))
            x0, wx = _axis01(xv, W)
            y0, wy = _axis01(yv, H)
            z0, wz = _axis01(zv, D)
            rbase = (bid << 18) + (z0 << 12) + (y0 << 6) + x0
            for k in range(8):
                idx_v[s, k, pl.ds(g * L, L)] = rbase + _CORNER_OFFS[k]
            wy1 = wy
            wy0 = 1.0 - wy
            wz1 = wz
            wz0 = 1.0 - wz
            wgt_v[s, pl.ds(0 * CV + g * L, L)] = wx
            wgt_v[s, pl.ds(1 * CV + g * L, L)] = wz0 * wy0
            wgt_v[s, pl.ds(2 * CV + g * L, L)] = wz0 * wy1
            wgt_v[s, pl.ds(3 * CV + g * L, L)] = wz1 * wy0
            wgt_v[s, pl.ds(4 * CV + g * L, L)] = wz1 * wy1
        for k in range(8):
            pltpu.async_copy(table.at[idx_v.at[s, k]], rows_v.at[s, k],
                             sem_g.at[s])

    def phase_b(j):
        s = j % 2
        ch = j * NW + wid
        for k in range(8):
            pltpu.make_async_copy(table.at[idx_v.at[s, k]],
                                  rows_v.at[s, k], sem_g.at[s]).wait()

        def lerp_group(g, _):
            wxv = wgt_v[s, pl.ds(0 * CV + g * L, L)]
            w00v = wgt_v[s, pl.ds(1 * CV + g * L, L)]
            w01v = wgt_v[s, pl.ds(2 * CV + g * L, L)]
            w10v = wgt_v[s, pl.ds(3 * CV + g * L, L)]
            w11v = wgt_v[s, pl.ds(4 * CV + g * L, L)]
            for lane in range(L):
                i = g * L + lane
                wx = wxv[lane]
                w00 = w00v[lane]
                w01 = w01v[lane]
                w10 = w10v[lane]
                w11 = w11v[lane]
                for h in range(C // L):
                    hs = pl.ds(h * L, L)
                    a = (w00 * rows_v[s, 0, i, hs] + w01 * rows_v[s, 1, i, hs]
                         + w10 * rows_v[s, 2, i, hs]
                         + w11 * rows_v[s, 3, i, hs])
                    bb = (w00 * rows_v[s, 4, i, hs] + w01 * rows_v[s, 5, i, hs]
                          + w10 * rows_v[s, 6, i, hs]
                          + w11 * rows_v[s, 7, i, hs])
                    outb_v[i, hs] = a + wx * (bb - a)
            return 0

        lax.fori_loop(0, CV // L, lerp_group, 0)
        pltpu.sync_copy(outb_v, out.at[pl.ds(ch * CV, CV)])

    fire_coords(0)

    def pipe_body(j, _):
        @pl.when(j < nch)
        def _():
            phase_a(j)

            @pl.when(j + 1 < nch)
            def _():
                fire_coords(j + 1)

        @pl.when((j >= 1) & (j <= nch))
        def _():
            phase_b(j - 1)
        return 0

    lax.fori_loop(0, NCH_LO + 2, pipe_body, 0)


_MESH = plsc.VectorSubcoreMesh(core_axis_name="c", subcore_axis_name="s")
_PARAMS = pltpu.CompilerParams(
    use_tc_tiling_on_sc=False, needs_layout_passes=False)


@jax.jit
def _run(voxel_features, vertices):
    table = jnp.transpose(voxel_features, (0, 2, 3, 4, 1)).reshape(VTOT, C)
    verts = vertices.reshape(NTOT, 3)
    out = pl.kernel(
        _sample_body,
        out_type=jax.ShapeDtypeStruct((NTOT, C), jnp.float32),
        mesh=_MESH,
        compiler_params=_PARAMS,
        scratch_types=[
            pltpu.VMEM((2, 8, CV), jnp.int32),
            pltpu.VMEM((2, 8, CV, C), jnp.float32),
            pltpu.VMEM((2, 5 * CV), jnp.float32),
            pltpu.VMEM((2, CV, 3), jnp.float32),
            pltpu.VMEM((CV, C), jnp.float32),
            pltpu.SemaphoreType.DMA((2,)),
            pltpu.SemaphoreType.DMA,
        ],
    )(table, verts)
    return out.reshape(B, N, C)


def kernel(voxel_features, vertices, pad_img_shape):
    del pad_img_shape
    return _run(voxel_features, vertices)


# speedup vs baseline: 1.5506x; 1.2955x over previous
import jax
import jax.numpy as jnp
from jax import lax
from jax.experimental import pallas as pl
from jax.experimental.pallas import tpu as pltpu
from jax.experimental.pallas import tpu_sc as plsc

B = 4
C = 32
D = H = W = 64
N = 100000
NTOT = B * N

NC = 2
NS = 16
NW = NC * NS
L = 16

DHW = D * H * W
VTOT = B * DHW

CV = 128
NCHT = NTOT // CV
NCH_LO = NCHT // NW
N_HI = NCHT - NCH_LO * NW

_CORNER_OFFS = (0, W, H * W, H * W + W, 1, W + 1, H * W + 1, H * W + W + 1)


def _axis01(v, extent):
    t = (v + 1.0) * 0.5 * (extent - 1)
    t = jnp.clip(t, 0.0, float(extent - 1))
    i0 = jnp.minimum(t.astype(jnp.int32), extent - 2)
    frac = t - i0.astype(jnp.float32)
    return i0, frac


def _sample_body(table, xs, ys, zs, out, idx_v, rows_v, wgt_v, crd_v, outb_v,
                 sem_g, sem_c):
    wid = lax.axis_index("s") * NC + lax.axis_index("c")
    nch = NCH_LO + jnp.where(wid < N_HI, 1, 0)

    def fire_coords(j):
        vbase = (j * NW + wid) * CV
        s = j % 2
        pltpu.async_copy(xs.at[pl.ds(vbase, CV)], crd_v.at[s, 0], sem_c)
        pltpu.async_copy(ys.at[pl.ds(vbase, CV)], crd_v.at[s, 1], sem_c)
        pltpu.async_copy(zs.at[pl.ds(vbase, CV)], crd_v.at[s, 2], sem_c)

    def wait_coords(j):
        s = j % 2
        vbase = (j * NW + wid) * CV
        pltpu.make_async_copy(xs.at[pl.ds(vbase, CV)], crd_v.at[s, 0],
                              sem_c).wait()
        pltpu.make_async_copy(ys.at[pl.ds(vbase, CV)], crd_v.at[s, 1],
                              sem_c).wait()
        pltpu.make_async_copy(zs.at[pl.ds(vbase, CV)], crd_v.at[s, 2],
                              sem_c).wait()

    def phase_a(j):
        s = j % 2
        ch = j * NW + wid
        vbase = ch * CV
        wait_coords(j)
        lanes = lax.iota(jnp.int32, L)
        for g in range(CV // L):
            xv = crd_v[s, 0, pl.ds(g * L, L)]
            yv = crd_v[s, 1, pl.ds(g * L, L)]
            zv = crd_v[s, 2, pl.ds(g * L, L)]
            gid = vbase + g * L + lanes
            one = jnp.ones((L,), jnp.int32)
            zero = jnp.zeros((L,), jnp.int32)
            bid = (jnp.where(gid >= N, one, zero)
                   + jnp.where(gid >= 2 * N, one, zero)
                   + jnp.where(gid >= 3 * N, one, zero))
            x0, wx = _axis01(xv, W)
            y0, wy = _axis01(yv, H)
            z0, wz = _axis01(zv, D)
            rbase = (bid << 18) + (z0 << 12) + (y0 << 6) + x0
            for k in range(8):
                idx_v[s, k, pl.ds(g * L, L)] = rbase + _CORNER_OFFS[k]
            wy1 = wy
            wy0 = 1.0 - wy
            wz1 = wz
            wz0 = 1.0 - wz
            wgt_v[s, pl.ds(0 * CV + g * L, L)] = wx
            wgt_v[s, pl.ds(1 * CV + g * L, L)] = wz0 * wy0
            wgt_v[s, pl.ds(2 * CV + g * L, L)] = wz0 * wy1
            wgt_v[s, pl.ds(3 * CV + g * L, L)] = wz1 * wy0
            wgt_v[s, pl.ds(4 * CV + g * L, L)] = wz1 * wy1
        for k in range(8):
            pltpu.async_copy(table.at[idx_v.at[s, k]], rows_v.at[s, k],
                             sem_g.at[s])

    def phase_b(j):
        s = j % 2
        ch = j * NW + wid
        for k in range(8):
            pltpu.make_async_copy(table.at[idx_v.at[s, k]],
                                  rows_v.at[s, k], sem_g.at[s]).wait()

        def lerp_group(g, _):
            wxv = wgt_v[s, pl.ds(0 * CV + g * L, L)]
            w00v = wgt_v[s, pl.ds(1 * CV + g * L, L)]
            w01v = wgt_v[s, pl.ds(2 * CV + g * L, L)]
            w10v = wgt_v[s, pl.ds(3 * CV + g * L, L)]
            w11v = wgt_v[s, pl.ds(4 * CV + g * L, L)]
            for lane in range(L):
                i = g * L + lane
                wx = wxv[lane]
                w00 = w00v[lane]
                w01 = w01v[lane]
                w10 = w10v[lane]
                w11 = w11v[lane]
                for h in range(C // L):
                    hs = pl.ds(h * L, L)
                    a = (w00 * rows_v[s, 0, i, hs] + w01 * rows_v[s, 1, i, hs]
                         + w10 * rows_v[s, 2, i, hs]
                         + w11 * rows_v[s, 3, i, hs])
                    bb = (w00 * rows_v[s, 4, i, hs] + w01 * rows_v[s, 5, i, hs]
                          + w10 * rows_v[s, 6, i, hs]
                          + w11 * rows_v[s, 7, i, hs])
                    outb_v[i, hs] = a + wx * (bb - a)
            return 0

        lax.fori_loop(0, CV // L, lerp_group, 0)
        pltpu.sync_copy(outb_v, out.at[pl.ds(ch * CV, CV)])

    fire_coords(0)

    def pipe_body(j, _):
        @pl.when(j < nch)
        def _():
            phase_a(j)

            @pl.when(j + 1 < nch)
            def _():
                fire_coords(j + 1)

        @pl.when((j >= 1) & (j <= nch))
        def _():
            phase_b(j - 1)
        return 0

    lax.fori_loop(0, NCH_LO + 2, pipe_body, 0)


_MESH = plsc.VectorSubcoreMesh(core_axis_name="c", subcore_axis_name="s")
_PARAMS = pltpu.CompilerParams(
    use_tc_tiling_on_sc=False, needs_layout_passes=False)


@jax.jit
def _run(voxel_features, vertices):
    table = jnp.transpose(voxel_features, (0, 2, 3, 4, 1)).reshape(VTOT, C)
    xs = vertices[:, :, 0].reshape(NTOT)
    ys = vertices[:, :, 1].reshape(NTOT)
    zs = vertices[:, :, 2].reshape(NTOT)
    out = pl.kernel(
        _sample_body,
        out_type=jax.ShapeDtypeStruct((NTOT, C), jnp.float32),
        mesh=_MESH,
        compiler_params=_PARAMS,
        scratch_types=[
            pltpu.VMEM((2, 8, CV), jnp.int32),
            pltpu.VMEM((2, 8, CV, C), jnp.float32),
            pltpu.VMEM((2, 5 * CV), jnp.float32),
            pltpu.VMEM((2, 3, CV), jnp.float32),
            pltpu.VMEM((CV, C), jnp.float32),
            pltpu.SemaphoreType.DMA((2,)),
            pltpu.SemaphoreType.DMA,
        ],
    )(table, xs, ys, zs)
    return out.reshape(B, N, C)


def kernel(voxel_features, vertices, pad_img_shape):
    del pad_img_shape
    return _run(voxel_features, vertices)
